# baseline (speedup 1.0000x reference)
import functools

import jax
import jax.numpy as jnp
from jax import lax
from jax.experimental import pallas as pl
from jax.experimental.pallas import tpu as pltpu
from jax.experimental.pallas import tpu_sc as plsc

E_NUM = 100000
R_NUM = 500
T_NUM = 365
D = 128
B = 4096

_S = 8
_L = 6656
_NBLK = _L // _S

_NC = 2
_NS = 16
_NW = _NC * _NS
_BPW = B // _NW
_CPW = 2 * _L // _NW


_NBPW = _NBLK // _NW
_GPW = _S * _NBPW


_NBPW = _NBLK // _NW
_GPW = _S * _NBPW


def _pipelined_gathers(seq, base, bufs, sems):
    handles = [None, None]
    handles[0] = pltpu.async_copy(seq[0][0].at[seq[0][2]], bufs[0], sems[0])
    for i in range(1, len(seq)):
        tbl, _, idxv = seq[i]
        handles[i % 2] = pltpu.async_copy(tbl.at[idxv], bufs[i % 2],
                                          sems[i % 2])
        handles[(i - 1) % 2].wait()
        pltpu.sync_copy(bufs[(i - 1) % 2],
                        seq[i - 1][1].at[pl.ds(base, _BPW)])
    handles[(len(seq) - 1) % 2].wait()
    pltpu.sync_copy(bufs[(len(seq) - 1) % 2],
                    seq[-1][1].at[pl.ds(base, _BPW)])


def _sc_a_body(co, head, target, ts, g8, order,
               ch, ct2, bts,
               ordv, hsv, tsv2, g8v, btsv, rows_a, rows_b,
               s0, s1, s6, sa, sb):
    wid = lax.axis_index("s") * _NC + lax.axis_index("c")
    base = wid * _BPW
    pltpu.sync_copy(order.at[pl.ds(base, _BPW)], ordv)
    h_hs = pltpu.async_copy(head.at[ordv], hsv, s0)
    h_ts = pltpu.async_copy(target.at[ordv], tsv2, s1)
    h_hs.wait()
    h_ts.wait()
    _pipelined_gathers([(co, ch, hsv), (co, ct2, tsv2)],
                       base, (rows_a, rows_b), (sa, sb))

    @pl.when(wid == 0)
    def _():
        pltpu.sync_copy(g8, g8v)
        pltpu.async_copy(ts.at[g8v], btsv, s6).wait()
        pltpu.sync_copy(btsv, bts)


def _sc_b_body(ex, ey, ez, rxt, ryt, rzt, head, target, rel, order,
               hx, hy, hz, ext, eyt, ezt, rx, ry, rz,
               ordv, hsv, tsv2, rsv, rows_a, rows_b,
               s0, s1, s2, sa, sb):
    wid = lax.axis_index("s") * _NC + lax.axis_index("c")
    base = wid * _BPW
    pltpu.sync_copy(order.at[pl.ds(base, _BPW)], ordv)
    h_hs = pltpu.async_copy(head.at[ordv], hsv, s0)
    h_ts = pltpu.async_copy(target.at[ordv], tsv2, s1)
    h_rs = pltpu.async_copy(rel.at[ordv], rsv, s2)
    h_hs.wait()
    h_ts.wait()
    h_rs.wait()
    seq = [(ex, hx, hsv), (ey, hy, hsv), (ez, hz, hsv),
           (ex, ext, tsv2), (ey, eyt, tsv2), (ez, ezt, tsv2),
           (rxt, rx, rsv), (ryt, ry, rsv), (rzt, rz, rsv)]
    _pipelined_gathers(seq, base, (rows_a, rows_b), (sa, sb))


def _sc_gather_a(cores, head, target, ts, g8, order):
    f32 = jnp.float32
    i32 = jnp.int32
    mesh = plsc.VectorSubcoreMesh(core_axis_name="c", subcore_axis_name="s")
    fn = pl.kernel(
        _sc_a_body,
        out_type=[jax.ShapeDtypeStruct((B + _S, D), f32)] * 2
        + [jax.ShapeDtypeStruct((_NBLK,), i32)],
        mesh=mesh,
        scratch_types=[
            pltpu.VMEM((_BPW,), i32),
            pltpu.VMEM((_BPW,), i32),
            pltpu.VMEM((_BPW,), i32),
            pltpu.VMEM((_NBLK,), i32),
            pltpu.VMEM((_NBLK,), i32),
            pltpu.VMEM((_BPW, D), f32),
            pltpu.VMEM((_BPW, D), f32),
        ] + [pltpu.SemaphoreType.DMA] * 5,
    )
    return fn(cores, head, target, ts, g8, order)


def _sc_gather_b(E_x, E_y, E_z, R_x, R_y, R_z, head, target, rel, order):
    f32 = jnp.float32
    i32 = jnp.int32
    mesh = plsc.VectorSubcoreMesh(core_axis_name="c", subcore_axis_name="s")
    fn = pl.kernel(
        _sc_b_body,
        out_type=[jax.ShapeDtypeStruct((B, D), f32)] * 9,
        mesh=mesh,
        scratch_types=[
            pltpu.VMEM((_BPW,), i32),
            pltpu.VMEM((_BPW,), i32),
            pltpu.VMEM((_BPW,), i32),
            pltpu.VMEM((_BPW,), i32),
            pltpu.VMEM((_BPW, D), f32),
            pltpu.VMEM((_BPW, D), f32),
        ] + [pltpu.SemaphoreType.DMA] * 5,
    )
    return fn(E_x, E_y, E_z, R_x, R_y, R_z, head, target, rel, order)


_UNROLL = 52


def _mv_body(bts_ref, jk_ref, time_ref, ch_ref, ct_ref, out1_ref, out2_ref):
    def body(i, carry):
        for u in range(_UNROLL):
            k = i * _UNROLL + u
            t = bts_ref[k]
            j = jk_ref[k]
            m = time_ref[t]
            h8 = ch_ref[pl.ds(j, _S), :]
            t8 = ct_ref[pl.ds(j, _S), :]
            v = jnp.concatenate([h8, t8], axis=0)
            o = lax.dot_general(v, m, (((1,), (1,)), ((), ())),
                                preferred_element_type=jnp.float32)
            out1_ref[pl.ds(j, _S), :] = o[0:_S]
            out2_ref[pl.ds(j, _S), :] = o[_S:2 * _S]
        return carry

    lax.fori_loop(0, _NBLK // _UNROLL, body, 0)


def _mv_call(bts, jk, time_mat, ch, ct2, interpret=False):
    f32 = jnp.float32
    return pl.pallas_call(
        _mv_body,
        out_shape=[jax.ShapeDtypeStruct((B + _S, D), f32)] * 2,
        in_specs=[
            pl.BlockSpec(memory_space=pltpu.SMEM),
            pl.BlockSpec(memory_space=pltpu.SMEM),
            pl.BlockSpec(memory_space=pltpu.VMEM),
            pl.BlockSpec(memory_space=pltpu.VMEM),
            pl.BlockSpec(memory_space=pltpu.VMEM),
        ],
        out_specs=[pl.BlockSpec(memory_space=pltpu.VMEM)] * 2,
        interpret=interpret,
    )(bts, jk, time_mat, ch, ct2)


def _fuse_body(hct_ref, ctm_ref, hx_ref, hy_ref, hz_ref,
               ext_ref, eyt_ref, ezt_ref, rx_ref, ry_ref, rz_ref,
               w_ref, b_ref, sx_ref, sy_ref, sz_ref, sct_ref):
    w = w_ref[:]
    b = b_ref[:]

    def bn(x):
        mean = jnp.mean(x, axis=0, keepdims=True)
        var = jnp.mean((x - mean) ** 2, axis=0, keepdims=True)
        return (x - mean) / jnp.sqrt(var + 1e-5) * w + b

    h_ct = bn(hct_ref[0:B, :])
    h_x = bn(hx_ref[:])
    h_y = bn(hy_ref[:])
    h_z = bn(hz_ref[:])

    r_x = rx_ref[:]
    r_y = ry_ref[:]
    r_z = rz_ref[:]
    length = jnp.sqrt(r_x * r_x + r_y * r_y + r_z * r_z)
    r_v_rate = jax.nn.sigmoid(length)
    inv_len = 1.0 / length
    r_x = r_x * inv_len
    r_y = r_y * inv_len
    r_z = r_z * inv_len
    gamma = 1.0 / jnp.sqrt(1.0 - r_v_rate * r_v_rate)
    gm1 = gamma - 1.0
    grv = gamma * r_v_rate

    t_ct = gamma * h_ct + grv * (r_x * h_x + r_y * h_y + r_z * h_z)
    t_x = (r_x * grv) * h_ct + (1.0 + r_x * r_x * gm1) * h_x \
        + (r_x * r_y * gm1) * h_y + (r_x * r_z * gm1) * h_z
    t_y = (r_y * grv) * h_ct + (r_x * r_y * gm1) * h_x \
        + (1.0 + r_y * r_y * gm1) * h_y + (r_z * r_y * gm1) * h_z
    t_z = (r_z * grv) * h_ct + (r_x * r_z * gm1) * h_x \
        + (r_y * r_z * gm1) * h_y + (1.0 + r_z * r_z * gm1) * h_z

    sx_ref[:] = jnp.sum(t_x * ext_ref[:], axis=1)
    sy_ref[:] = jnp.sum(t_y * eyt_ref[:], axis=1)
    sz_ref[:] = jnp.sum(t_z * ezt_ref[:], axis=1)
    sct_ref[:] = jnp.sum(t_ct * ctm_ref[0:B, :], axis=1)


def _fuse_call(hct, ctm, hx, hy, hz, ext, eyt, ezt, rx, ry, rz, bn_w, bn_b,
               interpret=False):
    f32 = jnp.float32
    return pl.pallas_call(
        _fuse_body,
        out_shape=[jax.ShapeDtypeStruct((B,), f32)] * 4,
        interpret=interpret,
    )(hct, ctm, hx, hy, hz, ext, eyt, ezt, rx, ry, rz, bn_w, bn_b)


def kernel(head, rel, timestamp, target, E_x, E_y, E_z, cores,
           R_x, R_y, R_z, time_mat, bn_w, bn_b):
    i32 = jnp.int32
    head = head.astype(i32)
    rel = rel.astype(i32)
    ts = timestamp.astype(i32)
    target = target.astype(i32)

    f32 = jnp.float32
    j = jnp.arange(B, dtype=i32)
    oh = (ts[:, None] == jnp.arange(T_NUM, dtype=i32)[None, :]).astype(f32)
    oh3 = oh.reshape(32, B // 32, T_NUM)
    ltri = jnp.tril(jnp.ones((B // 32, B // 32), f32), -1)
    hi = lax.Precision.HIGHEST
    within = jnp.einsum('ij,cjt->cit', ltri, oh3)
    chunk_tot = oh3.sum(axis=1)
    prefix = jnp.cumsum(chunk_tot, axis=0) - chunk_tot
    rank_within = jnp.round(jnp.sum((within + prefix[:, None, :]) * oh3,
                                    axis=-1).reshape(B))
    counts = chunk_tot.sum(axis=0)
    seg_start = jnp.cumsum(counts) - counts
    pad_counts = jnp.floor((counts + (_S - 1)) / _S) * _S
    pad_start = jnp.cumsum(pad_counts) - pad_counts
    rank = jnp.round(jnp.matmul(oh, seg_start, precision=hi)
                     + rank_within).astype(i32)
    slot = jnp.round(jnp.matmul(oh, pad_start, precision=hi)
                     + rank_within).astype(i32)
    order = jnp.zeros((B,), i32).at[rank].set(j)
    g = (jnp.arange(_L, dtype=i32) % B).at[slot].set(j)
    realmask = jnp.zeros((_L,), i32).at[slot].set(1)
    vb = realmask.reshape(_NBLK, _S).sum(axis=1)
    jk = (jnp.cumsum(vb) - vb).astype(i32)
    g8 = g.reshape(_NBLK, _S)[:, 0]

    ch, ct2, bts = _sc_gather_a(cores, head, target, ts, g8, order)
    (hx, hy, hz, ext, eyt, ezt, rx, ry, rz) = _sc_gather_b(
        E_x, E_y, E_z, R_x, R_y, R_z, head, target, rel, order)

    hct_p, ctm_p = _mv_call(bts, jk, time_mat, ch, ct2)
    sx, sy, sz, sct = _fuse_call(hct_p, ctm_p, hx, hy, hz, ext, eyt, ezt,
                                 rx, ry, rz, bn_w, bn_b)
    s4 = jnp.stack([sx, sy, sz, sct], axis=1)[rank]
    return s4[:, 0], s4[:, 1], s4[:, 2], s4[:, 3]


# speedup vs baseline: 1.0298x; 1.0298x over previous
import functools

import jax
import jax.numpy as jnp
from jax import lax
from jax.experimental import pallas as pl
from jax.experimental.pallas import tpu as pltpu
from jax.experimental.pallas import tpu_sc as plsc

E_NUM = 100000
R_NUM = 500
T_NUM = 365
D = 128
B = 4096

_S = 8
_L = 6656
_NBLK = _L // _S

_NC = 2
_NS = 16
_NW = _NC * _NS
_BPW = B // _NW
_CPW = 2 * _L // _NW


_NBPW = _NBLK // _NW
_GPW = _S * _NBPW


_NBPW = _NBLK // _NW
_GPW = _S * _NBPW


def _pipelined_gathers(seq, base, bufs, sems):
    handles = [None, None]
    handles[0] = pltpu.async_copy(seq[0][0].at[seq[0][2]], bufs[0], sems[0])
    for i in range(1, len(seq)):
        tbl, _, idxv = seq[i]
        handles[i % 2] = pltpu.async_copy(tbl.at[idxv], bufs[i % 2],
                                          sems[i % 2])
        handles[(i - 1) % 2].wait()
        pltpu.sync_copy(bufs[(i - 1) % 2],
                        seq[i - 1][1].at[pl.ds(base, _BPW)])
    handles[(len(seq) - 1) % 2].wait()
    pltpu.sync_copy(bufs[(len(seq) - 1) % 2],
                    seq[-1][1].at[pl.ds(base, _BPW)])


def _sc_gather_body(ex, ey, ez, co, rxt, ryt, rzt, head, target, rel,
                    g8, order, ts,
                    hx, hy, hz, ext, eyt, ezt, rx, ry, rz, ch, ct2, bts,
                    ordv, hsv, tsv2, rsv, g8v, btsv,
                    rows_a, rows_b,
                    s0, s1, s2, s6, sa, sb):
    wid = lax.axis_index("s") * _NC + lax.axis_index("c")
    base = wid * _BPW

    pltpu.sync_copy(order.at[pl.ds(base, _BPW)], ordv)
    h_hs = pltpu.async_copy(head.at[ordv], hsv, s0)
    h_ts = pltpu.async_copy(target.at[ordv], tsv2, s1)
    h_rs = pltpu.async_copy(rel.at[ordv], rsv, s2)
    h_hs.wait()
    h_ts.wait()
    h_rs.wait()
    seq = [(co, ch, hsv), (co, ct2, tsv2),
           (ex, hx, hsv), (ey, hy, hsv), (ez, hz, hsv),
           (ex, ext, tsv2), (ey, eyt, tsv2), (ez, ezt, tsv2),
           (rxt, rx, rsv), (ryt, ry, rsv), (rzt, rz, rsv)]
    _pipelined_gathers(seq, base, (rows_a, rows_b), (sa, sb))

    @pl.when(wid == 0)
    def _():
        pltpu.sync_copy(g8, g8v)
        pltpu.async_copy(ts.at[g8v], btsv, s6).wait()
        pltpu.sync_copy(btsv, bts)


def _sc_gather(E_x, E_y, E_z, cores, R_x, R_y, R_z, head, target, rel,
               g8, order, ts):
    f32 = jnp.float32
    i32 = jnp.int32
    out_type = (
        [jax.ShapeDtypeStruct((B, D), f32)] * 9
        + [jax.ShapeDtypeStruct((B + _S, D), f32)] * 2
        + [jax.ShapeDtypeStruct((_NBLK,), i32)]
    )
    mesh = plsc.VectorSubcoreMesh(core_axis_name="c", subcore_axis_name="s")
    fn = pl.kernel(
        _sc_gather_body,
        out_type=out_type,
        mesh=mesh,
        scratch_types=[
            pltpu.VMEM((_BPW,), i32),
            pltpu.VMEM((_BPW,), i32),
            pltpu.VMEM((_BPW,), i32),
            pltpu.VMEM((_BPW,), i32),
            pltpu.VMEM((_NBLK,), i32),
            pltpu.VMEM((_NBLK,), i32),
            pltpu.VMEM((_BPW, D), f32),
            pltpu.VMEM((_BPW, D), f32),
        ] + [pltpu.SemaphoreType.DMA] * 6,
    )
    return fn(E_x, E_y, E_z, cores, R_x, R_y, R_z, head, target, rel,
              g8, order, ts)


_UNROLL = 52


def _mv_body(bts_ref, jk_ref, time_ref, ch_ref, ct_ref, out1_ref, out2_ref):
    def body(i, carry):
        for u in range(_UNROLL):
            k = i * _UNROLL + u
            t = bts_ref[k]
            j = jk_ref[k]
            m = time_ref[t]
            h8 = ch_ref[pl.ds(j, _S), :]
            t8 = ct_ref[pl.ds(j, _S), :]
            v = jnp.concatenate([h8, t8], axis=0)
            o = lax.dot_general(v, m, (((1,), (1,)), ((), ())),
                                preferred_element_type=jnp.float32)
            out1_ref[pl.ds(j, _S), :] = o[0:_S]
            out2_ref[pl.ds(j, _S), :] = o[_S:2 * _S]
        return carry

    lax.fori_loop(0, _NBLK // _UNROLL, body, 0)


def _mv_call(bts, jk, time_mat, ch, ct2, interpret=False):
    f32 = jnp.float32
    return pl.pallas_call(
        _mv_body,
        out_shape=[jax.ShapeDtypeStruct((B + _S, D), f32)] * 2,
        in_specs=[
            pl.BlockSpec(memory_space=pltpu.SMEM),
            pl.BlockSpec(memory_space=pltpu.SMEM),
            pl.BlockSpec(memory_space=pltpu.VMEM),
            pl.BlockSpec(memory_space=pltpu.VMEM),
            pl.BlockSpec(memory_space=pltpu.VMEM),
        ],
        out_specs=[pl.BlockSpec(memory_space=pltpu.VMEM)] * 2,
        interpret=interpret,
    )(bts, jk, time_mat, ch, ct2)


def _fuse_body(hct_ref, ctm_ref, hx_ref, hy_ref, hz_ref,
               ext_ref, eyt_ref, ezt_ref, rx_ref, ry_ref, rz_ref,
               w_ref, b_ref, sx_ref, sy_ref, sz_ref, sct_ref):
    w = w_ref[:]
    b = b_ref[:]

    def bn(x):
        mean = jnp.mean(x, axis=0, keepdims=True)
        var = jnp.mean((x - mean) ** 2, axis=0, keepdims=True)
        return (x - mean) / jnp.sqrt(var + 1e-5) * w + b

    h_ct = bn(hct_ref[0:B, :])
    h_x = bn(hx_ref[:])
    h_y = bn(hy_ref[:])
    h_z = bn(hz_ref[:])

    r_x = rx_ref[:]
    r_y = ry_ref[:]
    r_z = rz_ref[:]
    length = jnp.sqrt(r_x * r_x + r_y * r_y + r_z * r_z)
    r_v_rate = jax.nn.sigmoid(length)
    inv_len = 1.0 / length
    r_x = r_x * inv_len
    r_y = r_y * inv_len
    r_z = r_z * inv_len
    gamma = 1.0 / jnp.sqrt(1.0 - r_v_rate * r_v_rate)
    gm1 = gamma - 1.0
    grv = gamma * r_v_rate

    t_ct = gamma * h_ct + grv * (r_x * h_x + r_y * h_y + r_z * h_z)
    t_x = (r_x * grv) * h_ct + (1.0 + r_x * r_x * gm1) * h_x \
        + (r_x * r_y * gm1) * h_y + (r_x * r_z * gm1) * h_z
    t_y = (r_y * grv) * h_ct + (r_x * r_y * gm1) * h_x \
        + (1.0 + r_y * r_y * gm1) * h_y + (r_z * r_y * gm1) * h_z
    t_z = (r_z * grv) * h_ct + (r_x * r_z * gm1) * h_x \
        + (r_y * r_z * gm1) * h_y + (1.0 + r_z * r_z * gm1) * h_z

    sx_ref[:] = jnp.sum(t_x * ext_ref[:], axis=1)
    sy_ref[:] = jnp.sum(t_y * eyt_ref[:], axis=1)
    sz_ref[:] = jnp.sum(t_z * ezt_ref[:], axis=1)
    sct_ref[:] = jnp.sum(t_ct * ctm_ref[0:B, :], axis=1)


def _fuse_call(hct, ctm, hx, hy, hz, ext, eyt, ezt, rx, ry, rz, bn_w, bn_b,
               interpret=False):
    f32 = jnp.float32
    return pl.pallas_call(
        _fuse_body,
        out_shape=[jax.ShapeDtypeStruct((B,), f32)] * 4,
        interpret=interpret,
    )(hct, ctm, hx, hy, hz, ext, eyt, ezt, rx, ry, rz, bn_w, bn_b)


def kernel(head, rel, timestamp, target, E_x, E_y, E_z, cores,
           R_x, R_y, R_z, time_mat, bn_w, bn_b):
    i32 = jnp.int32
    head = head.astype(i32)
    rel = rel.astype(i32)
    ts = timestamp.astype(i32)
    target = target.astype(i32)

    f32 = jnp.float32
    j = jnp.arange(B, dtype=i32)
    oh = (ts[:, None] == jnp.arange(T_NUM, dtype=i32)[None, :]).astype(f32)
    oh3 = oh.reshape(32, B // 32, T_NUM)
    ltri = jnp.tril(jnp.ones((B // 32, B // 32), f32), -1)
    hi = lax.Precision.HIGHEST
    within = jnp.einsum('ij,cjt->cit', ltri, oh3)
    chunk_tot = oh3.sum(axis=1)
    prefix = jnp.cumsum(chunk_tot, axis=0) - chunk_tot
    rank_within = jnp.round(jnp.sum((within + prefix[:, None, :]) * oh3,
                                    axis=-1).reshape(B))
    counts = chunk_tot.sum(axis=0)
    seg_start = jnp.cumsum(counts) - counts
    pad_counts = jnp.floor((counts + (_S - 1)) / _S) * _S
    pad_start = jnp.cumsum(pad_counts) - pad_counts
    rank = jnp.round(jnp.matmul(oh, seg_start, precision=hi)
                     + rank_within).astype(i32)
    slot = jnp.round(jnp.matmul(oh, pad_start, precision=hi)
                     + rank_within).astype(i32)
    order = jnp.zeros((B,), i32).at[rank].set(j)
    g = (jnp.arange(_L, dtype=i32) % B).at[slot].set(j)
    realmask = jnp.zeros((_L,), i32).at[slot].set(1)
    vb = realmask.reshape(_NBLK, _S).sum(axis=1)
    jk = (jnp.cumsum(vb) - vb).astype(i32)
    g8 = g.reshape(_NBLK, _S)[:, 0]

    (hx, hy, hz, ext, eyt, ezt, rx, ry, rz, ch, ct2, bts) = _sc_gather(
        E_x, E_y, E_z, cores, R_x, R_y, R_z, head, target, rel,
        g8, order, ts)

    hct_p, ctm_p = _mv_call(bts, jk, time_mat, ch, ct2)
    sx, sy, sz, sct = _fuse_call(hct_p, ctm_p, hx, hy, hz, ext, eyt, ezt,
                                 rx, ry, rz, bn_w, bn_b)
    s4 = jnp.stack([sx, sy, sz, sct], axis=1)[rank]
    return s4[:, 0], s4[:, 1], s4[:, 2], s4[:, 3]
